# baseline (device time: 23329 ns/iter reference)
import jax
import jax.numpy as jnp
from jax import lax
from jax.experimental import pallas as pl
from jax.experimental.pallas import tpu as pltpu

N_DEV = 8
BLK = 64
N_STAGE = 2


def kernel(x, Wq, K_ext, V_ext, Wo):
    B, Sq, Dm = x.shape
    _, Skv_loc, Hq, Dh = K_ext.shape
    HD = Hq * Dh
    R = B * Sq
    HPS = Hq // N_STAGE
    CW = HPS * Dh
    bf16 = jnp.bfloat16

    x2 = x.reshape(R, Dm)
    K3 = K_ext.reshape(R, Hq, Dh)
    V3 = V_ext.reshape(R, Hq, Dh)

    def body(x_ref, wq_ref, k_ref, v_ref, wo_ref, out_ref,
             kbf_ref, vbf_ref, kvin_ref, ctx_ref,
             send1, recv1, send2, recv2):
        me = lax.axis_index("i")
        row0 = me * BLK
        myrows = pl.ds(row0, BLK)

        barrier = pltpu.get_barrier_semaphore()
        for j in range(1, N_DEV):
            peer = lax.rem(me + j, N_DEV)
            pl.semaphore_signal(barrier, inc=1, device_id=(peer,),
                                device_id_type=pl.DeviceIdType.MESH)

        def relayout(r, heads):
            rows = pl.ds(r * BLK, BLK)
            for src, dst in ((k_ref, kbf_ref), (v_ref, vbf_ref)):
                for h in heads:
                    dst[rows, h * Dh:(h + 1) * Dh] = (
                        src[rows, h, :].astype(bf16))

        relayout(me, range(Hq))
        kvin_ref[me, 0] = kbf_ref[myrows, :]
        kvin_ref[me, 1] = vbf_ref[myrows, :]

        pl.semaphore_wait(barrier, N_DEV - 1)

        all_rdmas = []
        qrow = None
        for s in range(N_STAGE):
            cs = slice(s * CW, (s + 1) * CW)
            heads = range(s * HPS, (s + 1) * HPS)

            for j in range(1, N_DEV):
                peer = lax.rem(me + j, N_DEV)
                relayout(peer, heads)
                for t, src in ((0, kbf_ref), (1, vbf_ref)):
                    rdma = pltpu.make_async_remote_copy(
                        src_ref=src.at[pl.ds(peer * BLK, BLK), cs],
                        dst_ref=kvin_ref.at[me, t, :, cs],
                        send_sem=send1.at[s * 2 * (N_DEV - 1) + 2 * (j - 1) + t],
                        recv_sem=recv1.at[me, t, s],
                        device_id=(peer,),
                        device_id_type=pl.DeviceIdType.MESH,
                    )
                    rdma.start()
                    all_rdmas.append(rdma)

            if qrow is None:
                qrow = lax.dot_general(x_ref[myrows, :].astype(bf16),
                                       wq_ref[...].astype(bf16),
                                       (((1,), (0,)), ((), ())),
                                       preferred_element_type=jnp.float32
                                       ).astype(bf16)

            for j in range(1, N_DEV):
                origin = lax.rem(me + j, N_DEV)
                for t in (0, 1):
                    recv = pltpu.make_async_remote_copy(
                        src_ref=kvin_ref.at[origin, t, :, cs],
                        dst_ref=kvin_ref.at[origin, t, :, cs],
                        send_sem=send1.at[s * 2 * (N_DEV - 1) + 2 * (j - 1) + t],
                        recv_sem=recv1.at[origin, t, s],
                        device_id=(me,), device_id_type=pl.DeviceIdType.MESH,
                    )
                    recv.wait_recv()

            for h in heads:
                hs = slice(h * Dh, (h + 1) * Dh)
                qh = qrow[:, hs]
                scores = []
                for c in range(N_DEV):
                    sc = lax.dot_general(qh, kvin_ref[c, 0, :, hs],
                                         (((1,), (1,)), ((), ())),
                                         preferred_element_type=jnp.float32)
                    scores.append(sc)
                S = jnp.concatenate(scores, axis=-1) * 0.125
                m = jnp.max(S, axis=-1, keepdims=True)
                w = jnp.exp(S - m)
                P = (w / jnp.sum(w, axis=-1, keepdims=True)).astype(bf16)
                acc = jnp.zeros((BLK, Dh), jnp.float32)
                for c in range(N_DEV):
                    acc = acc + lax.dot_general(
                        P[:, c * BLK:(c + 1) * BLK], kvin_ref[c, 1, :, hs],
                        (((1,), (0,)), ((), ())),
                        preferred_element_type=jnp.float32)
                ctx_ref[myrows, hs] = acc.astype(bf16)

            for j in range(1, N_DEV):
                peer = lax.rem(me + j, N_DEV)
                rdma = pltpu.make_async_remote_copy(
                    src_ref=ctx_ref.at[myrows, cs],
                    dst_ref=ctx_ref.at[myrows, cs],
                    send_sem=send2.at[s * (N_DEV - 1) + (j - 1)],
                    recv_sem=recv2.at[me, s],
                    device_id=(peer,), device_id_type=pl.DeviceIdType.MESH,
                )
                rdma.start()
                all_rdmas.append(rdma)

        for j in range(1, N_DEV):
            origin = lax.rem(me + j, N_DEV)
            for s in range(N_STAGE):
                cs = slice(s * CW, (s + 1) * CW)
                recv = pltpu.make_async_remote_copy(
                    src_ref=ctx_ref.at[pl.ds(origin * BLK, BLK), cs],
                    dst_ref=ctx_ref.at[pl.ds(origin * BLK, BLK), cs],
                    send_sem=send2.at[s * (N_DEV - 1) + (j - 1)],
                    recv_sem=recv2.at[origin, s],
                    device_id=(me,), device_id_type=pl.DeviceIdType.MESH,
                )
                recv.wait_recv()
        for rdma in all_rdmas:
            rdma.wait_send()

        out = lax.dot_general(ctx_ref[...], wo_ref[...].astype(bf16),
                              (((1,), (0,)), ((), ())),
                              preferred_element_type=jnp.float32)
        out_ref[...] = out.astype(bf16).reshape(B, Sq, Dm)

    return pl.pallas_call(
        body,
        out_shape=jax.ShapeDtypeStruct((B, Sq, Dm), bf16),
        in_specs=[pl.BlockSpec(memory_space=pltpu.VMEM)] * 5,
        out_specs=pl.BlockSpec(memory_space=pltpu.VMEM),
        scratch_shapes=[
            pltpu.VMEM((R, HD), bf16),
            pltpu.VMEM((R, HD), bf16),
            pltpu.VMEM((N_DEV, 2, BLK, HD), bf16),
            pltpu.VMEM((R, HD), bf16),
            pltpu.SemaphoreType.DMA((N_STAGE * 2 * (N_DEV - 1),)),
            pltpu.SemaphoreType.DMA((N_DEV, 2, N_STAGE)),
            pltpu.SemaphoreType.DMA((N_STAGE * (N_DEV - 1),)),
            pltpu.SemaphoreType.DMA((N_DEV, N_STAGE)),
        ],
        compiler_params=pltpu.CompilerParams(collective_id=0),
    )(x2, Wq, K3, V3, Wo)


# device time: 18996 ns/iter; 1.2281x vs baseline; 1.2281x over previous
import jax
import jax.numpy as jnp
from jax import lax
from jax.experimental import pallas as pl
from jax.experimental.pallas import tpu as pltpu

N_DEV = 8
BLK = 64


def kernel(x, Wq, K_ext, V_ext, Wo):
    B, Sq, Dm = x.shape
    _, Skv_loc, Hq, Dh = K_ext.shape
    HD = Hq * Dh
    R = B * Sq
    bf16 = jnp.bfloat16

    x2 = x.reshape(R, Dm)
    K3 = K_ext.reshape(R, Hq, Dh)
    V3 = V_ext.reshape(R, Hq, Dh)

    def body(x_ref, wq_ref, k_ref, v_ref, wo_ref, out_ref,
             kbf_ref, vbf_ref, kvin_ref, sc_ref, ctx_ref,
             send1, recv1, send2, recv2):
        me = lax.axis_index("i")
        row0 = me * BLK
        myrows = pl.ds(row0, BLK)

        barrier = pltpu.get_barrier_semaphore()
        for j in range(1, N_DEV):
            peer = lax.rem(me + j, N_DEV)
            pl.semaphore_signal(barrier, inc=1, device_id=(peer,),
                                device_id_type=pl.DeviceIdType.MESH)

        def relayout(r, t):
            rows = pl.ds(r * BLK, BLK)
            src, dst = ((k_ref, kbf_ref), (v_ref, vbf_ref))[t]
            for h in range(Hq):
                dst[rows, h * Dh:(h + 1) * Dh] = src[rows, h, :].astype(bf16)

        relayout(me, 0)
        relayout(me, 1)
        kvin_ref[me, 0] = kbf_ref[myrows, :]
        kvin_ref[me, 1] = vbf_ref[myrows, :]

        pl.semaphore_wait(barrier, N_DEV - 1)

        all_rdmas = []
        for t, src in ((0, kbf_ref), (1, vbf_ref)):
            for j in range(1, N_DEV):
                peer = lax.rem(me + j, N_DEV)
                relayout(peer, t)
                rdma = pltpu.make_async_remote_copy(
                    src_ref=src.at[pl.ds(peer * BLK, BLK), :],
                    dst_ref=kvin_ref.at[me, t],
                    send_sem=send1.at[t * (N_DEV - 1) + (j - 1)],
                    recv_sem=recv1.at[me, t],
                    device_id=(peer,),
                    device_id_type=pl.DeviceIdType.MESH,
                )
                rdma.start()
                all_rdmas.append(rdma)

        qrow = lax.dot_general(x_ref[myrows, :].astype(bf16),
                               wq_ref[...].astype(bf16),
                               (((1,), (0,)), ((), ())),
                               preferred_element_type=jnp.float32
                               ).astype(bf16)

        for h in range(Hq):
            hs = slice(h * Dh, (h + 1) * Dh)
            sc_ref[h, me] = lax.dot_general(
                qrow[:, hs], kvin_ref[me, 0, :, hs],
                (((1,), (1,)), ((), ())),
                preferred_element_type=jnp.float32)

        for j in range(1, N_DEV):
            origin = lax.rem(me + N_DEV - j, N_DEV)
            recv = pltpu.make_async_remote_copy(
                src_ref=kvin_ref.at[origin, 0], dst_ref=kvin_ref.at[origin, 0],
                send_sem=send1.at[j - 1], recv_sem=recv1.at[origin, 0],
                device_id=(me,), device_id_type=pl.DeviceIdType.MESH,
            )
            recv.wait_recv()
            for h in range(Hq):
                hs = slice(h * Dh, (h + 1) * Dh)
                sc_ref[h, origin] = lax.dot_general(
                    qrow[:, hs], kvin_ref[origin, 0, :, hs],
                    (((1,), (1,)), ((), ())),
                    preferred_element_type=jnp.float32)

        for j in range(1, N_DEV):
            origin = lax.rem(me + N_DEV - j, N_DEV)
            recv = pltpu.make_async_remote_copy(
                src_ref=kvin_ref.at[origin, 1], dst_ref=kvin_ref.at[origin, 1],
                send_sem=send1.at[(N_DEV - 1) + (j - 1)],
                recv_sem=recv1.at[origin, 1],
                device_id=(me,), device_id_type=pl.DeviceIdType.MESH,
            )
            recv.wait_recv()

        for h in range(Hq):
            hs = slice(h * Dh, (h + 1) * Dh)
            S = sc_ref[h] * 0.125
            m = jnp.max(S, axis=(0, 2), keepdims=True)
            w = jnp.exp(S - m)
            P = (w / jnp.sum(w, axis=(0, 2), keepdims=True)).astype(bf16)
            acc = jnp.zeros((BLK, Dh), jnp.float32)
            for c in range(N_DEV):
                acc = acc + lax.dot_general(
                    P[c], kvin_ref[c, 1, :, hs],
                    (((1,), (0,)), ((), ())),
                    preferred_element_type=jnp.float32)
            ctx_ref[myrows, hs] = acc.astype(bf16)

        for j in range(1, N_DEV):
            peer = lax.rem(me + j, N_DEV)
            rdma = pltpu.make_async_remote_copy(
                src_ref=ctx_ref.at[myrows, :],
                dst_ref=ctx_ref.at[myrows, :],
                send_sem=send2.at[j - 1], recv_sem=recv2.at[me],
                device_id=(peer,), device_id_type=pl.DeviceIdType.MESH,
            )
            rdma.start()
            all_rdmas.append(rdma)
        for j in range(1, N_DEV):
            origin = lax.rem(me + j, N_DEV)
            recv = pltpu.make_async_remote_copy(
                src_ref=ctx_ref.at[pl.ds(origin * BLK, BLK), :],
                dst_ref=ctx_ref.at[pl.ds(origin * BLK, BLK), :],
                send_sem=send2.at[j - 1], recv_sem=recv2.at[origin],
                device_id=(me,), device_id_type=pl.DeviceIdType.MESH,
            )
            recv.wait_recv()
        for rdma in all_rdmas:
            rdma.wait_send()

        out = lax.dot_general(ctx_ref[...], wo_ref[...].astype(bf16),
                              (((1,), (0,)), ((), ())),
                              preferred_element_type=jnp.float32)
        out_ref[...] = out.astype(bf16).reshape(B, Sq, Dm)

    return pl.pallas_call(
        body,
        out_shape=jax.ShapeDtypeStruct((B, Sq, Dm), bf16),
        in_specs=[pl.BlockSpec(memory_space=pltpu.VMEM)] * 5,
        out_specs=pl.BlockSpec(memory_space=pltpu.VMEM),
        scratch_shapes=[
            pltpu.VMEM((R, HD), bf16),
            pltpu.VMEM((R, HD), bf16),
            pltpu.VMEM((N_DEV, 2, BLK, HD), bf16),
            pltpu.VMEM((Hq, N_DEV, BLK, BLK), jnp.float32),
            pltpu.VMEM((R, HD), bf16),
            pltpu.SemaphoreType.DMA((2 * (N_DEV - 1),)),
            pltpu.SemaphoreType.DMA((N_DEV, 2)),
            pltpu.SemaphoreType.DMA((N_DEV - 1,)),
            pltpu.SemaphoreType.DMA((N_DEV,)),
        ],
        compiler_params=pltpu.CompilerParams(collective_id=0),
    )(x2, Wq, K3, V3, Wo)
